# Initial kernel scaffold; baseline (speedup 1.0000x reference)
import functools

import jax
import jax.numpy as jnp
from jax import lax
from jax.experimental import pallas as pl
from jax.experimental.pallas import tpu as pltpu
from jax.experimental.pallas import tpu_sc as plsc

S = 10000
N = 320000
D = 128

NC = 2
NS = 16
NW = NC * NS
ROWS_PER_W = N // NW
BLK = 125
NBLK = ROWS_PER_W // BLK
SEG_PER_TILE = S // NS


def _sc_segment_partials(v3, idx3, zeros):
    mesh = plsc.VectorSubcoreMesh(core_axis_name="c", subcore_axis_name="s")

    @functools.partial(
        pl.kernel,
        out_type=jax.ShapeDtypeStruct((NC, S, D), jnp.float32),
        mesh=mesh,
        scratch_types=[
            pltpu.VMEM((NBLK, BLK), jnp.int32),
            pltpu.VMEM((BLK, D), jnp.float32),
            pltpu.VMEM_SHARED((S, D), jnp.float32),
        ],
    )
    def body(v_hbm, idx_hbm, zero_hbm, out_hbm, idx_v, rows_v, acc_sh):
        c = lax.axis_index("c")
        s = lax.axis_index("s")
        wid = c * NS + s

        pltpu.sync_copy(zero_hbm, acc_sh.at[pl.ds(s * SEG_PER_TILE, SEG_PER_TILE)])
        pltpu.sync_copy(idx_hbm.at[wid], idx_v)
        plsc.subcore_barrier()

        def step(j, carry):
            pltpu.sync_copy(v_hbm.at[wid, j], rows_v)
            pltpu.sync_copy(rows_v, acc_sh.at[idx_v.at[j]], add=True)
            return carry

        lax.fori_loop(0, NBLK, step, 0)
        plsc.subcore_barrier()

        sl = pl.ds(s * SEG_PER_TILE, SEG_PER_TILE)
        pltpu.sync_copy(acc_sh.at[sl], out_hbm.at[c, sl])

    return body(v3, idx3, zeros)


def _tc_combine(u, parts):
    blk = 1000

    def body(u_ref, p_ref, o_ref):
        o_ref[...] = u_ref[...] + p_ref[0] + p_ref[1]

    return pl.pallas_call(
        body,
        grid=(S // blk,),
        in_specs=[
            pl.BlockSpec((blk, D), lambda i: (i, 0)),
            pl.BlockSpec((NC, blk, D), lambda i: (0, i, 0)),
        ],
        out_specs=pl.BlockSpec((blk, D), lambda i: (i, 0)),
        out_shape=jax.ShapeDtypeStruct((S, D), jnp.float32),
    )(u, parts)


def kernel(u, v, batch):
    idx3 = batch.astype(jnp.int32).reshape(NW, NBLK, BLK)
    v3 = v.reshape(NW, NBLK, BLK, D)
    zeros = jnp.zeros((SEG_PER_TILE, D), jnp.float32)
    parts = _sc_segment_partials(v3, idx3, zeros)
    return _tc_combine(u, parts)


# speedup vs baseline: 4.5018x; 4.5018x over previous
import functools

import jax
import jax.numpy as jnp
from jax import lax
from jax.experimental import pallas as pl
from jax.experimental.pallas import tpu as pltpu
from jax.experimental.pallas import tpu_sc as plsc

S = 10000
N = 320000
D = 128

NC = 2
NS = 16
NW = NC * NS
ROWS_PER_W = N // NW
BLK = 80
NBLK = ROWS_PER_W // BLK
S_PAD = 10240
SEG_PER_TILE = S_PAD // NS


def _sc_segment_partials(v3, idx3, zeros):
    mesh = plsc.VectorSubcoreMesh(core_axis_name="c", subcore_axis_name="s")

    @functools.partial(
        pl.kernel,
        out_type=jax.ShapeDtypeStruct((NC, S_PAD, D), jnp.float32),
        mesh=mesh,
        scratch_types=[
            pltpu.VMEM((NBLK, BLK), jnp.int32),
            pltpu.VMEM((BLK, D), jnp.float32),
            pltpu.VMEM_SHARED((S_PAD, D), jnp.float32),
        ],
    )
    def body(v_hbm, idx_hbm, zero_hbm, out_hbm, idx_v, rows_v, acc_sh):
        c = lax.axis_index("c")
        s = lax.axis_index("s")
        wid = c * NS + s

        pltpu.sync_copy(zero_hbm, acc_sh.at[pl.ds(s * SEG_PER_TILE, SEG_PER_TILE)])
        pltpu.sync_copy(idx_hbm.at[wid], idx_v)
        plsc.subcore_barrier()

        def step(j, carry):
            pltpu.sync_copy(v_hbm.at[wid, j], rows_v)
            pltpu.sync_copy(rows_v, acc_sh.at[idx_v.at[j]], add=True)
            return carry

        lax.fori_loop(0, NBLK, step, 0)
        plsc.subcore_barrier()

        sl = pl.ds(s * SEG_PER_TILE, SEG_PER_TILE)
        pltpu.sync_copy(acc_sh.at[sl], out_hbm.at[c, sl])

    return body(v3, idx3, zeros)


def _tc_combine(u, parts):
    blk = 1000

    def body(u_ref, p_ref, o_ref):
        o_ref[...] = u_ref[...] + p_ref[0] + p_ref[1]

    return pl.pallas_call(
        body,
        grid=(S // blk,),
        in_specs=[
            pl.BlockSpec((blk, D), lambda i: (i, 0)),
            pl.BlockSpec((NC, blk, D), lambda i: (0, i, 0)),
        ],
        out_specs=pl.BlockSpec((blk, D), lambda i: (i, 0)),
        out_shape=jax.ShapeDtypeStruct((S, D), jnp.float32),
    )(u, parts)


def kernel(u, v, batch):
    idx3 = batch.astype(jnp.int32).reshape(NW, NBLK, BLK)
    v3 = v.reshape(NW, NBLK, BLK, D)
    zeros = jnp.zeros((SEG_PER_TILE, D), jnp.float32)
    parts = _sc_segment_partials(v3, idx3, zeros)
    return _tc_combine(u, parts)
